# baseline (device time: 27266 ns/iter reference)
import jax
import jax.numpy as jnp
from jax import lax
from jax.experimental import pallas as pl
from jax.experimental.pallas import tpu as pltpu

BM = 512


def kernel(x, dy, gamma):
    m, d = x.shape
    n_blocks = m // BM

    def body(x_ref, dy_ref, out_ref, acc_ref, comm_ref, send_sem, recv_sem):
        step = pl.program_id(0)

        xb = x_ref[...]
        dyb = dy_ref[...]
        mu = jnp.mean(xb, axis=1, keepdims=True)
        var = jnp.mean(xb * xb, axis=1, keepdims=True) - mu * mu
        rstd = lax.rsqrt(var + 1e-5)
        xhat = (xb - mu) * rstd
        dgamma_p = jnp.sum(dyb * xhat, axis=0, keepdims=True)
        dbeta_p = jnp.sum(dyb, axis=0, keepdims=True)
        part = jnp.concatenate([dgamma_p, dbeta_p], axis=0)

        @pl.when(step == 0)
        def _():
            acc_ref[...] = part

        @pl.when(step != 0)
        def _():
            acc_ref[...] = acc_ref[...] + part

        @pl.when(step == n_blocks - 1)
        def _():
            my_x = lax.axis_index("x")
            my_y = lax.axis_index("y")
            my_z = lax.axis_index("z")
            partner = (my_x, 1 - my_y, my_z)

            barrier = pltpu.get_barrier_semaphore()
            pl.semaphore_signal(
                barrier, inc=1,
                device_id=partner, device_id_type=pl.DeviceIdType.MESH,
            )
            pl.semaphore_wait(barrier, 1)

            comm_ref[0, ...] = acc_ref[...]
            rdma = pltpu.make_async_remote_copy(
                src_ref=comm_ref.at[0],
                dst_ref=comm_ref.at[1],
                send_sem=send_sem,
                recv_sem=recv_sem,
                device_id=partner,
                device_id_type=pl.DeviceIdType.MESH,
            )
            rdma.start()
            rdma.wait()
            out_ref[...] = acc_ref[...] + comm_ref[1, ...]

    return pl.pallas_call(
        body,
        grid=(n_blocks,),
        in_specs=[
            pl.BlockSpec((BM, d), lambda i: (i, 0)),
            pl.BlockSpec((BM, d), lambda i: (i, 0)),
        ],
        out_specs=pl.BlockSpec((2, d), lambda i: (0, 0)),
        out_shape=jax.ShapeDtypeStruct((2, d), jnp.float32),
        scratch_shapes=[
            pltpu.VMEM((2, d), jnp.float32),
            pltpu.VMEM((2, 2, d), jnp.float32),
            pltpu.SemaphoreType.DMA,
            pltpu.SemaphoreType.DMA,
        ],
        compiler_params=pltpu.CompilerParams(
            collective_id=0,
            dimension_semantics=("arbitrary",),
        ),
    )(x, dy)


# device time: 13593 ns/iter; 2.0059x vs baseline; 2.0059x over previous
import jax
import jax.numpy as jnp
from jax import lax
from jax.experimental import pallas as pl
from jax.experimental.pallas import tpu as pltpu

X, Y, Z = 2, 2, 4
N_DEV = X * Y * Z
N_REP = X * Z


def kernel(x, dy, gamma):
    m, d = x.shape
    rows = m // N_REP

    def body(x_hbm, dy_hbm, out_ref, xv, dyv, acc_ref, comm_ref,
             in_sems, loc_sem, send_sems, recv_sems):
        my_x = lax.axis_index("x")
        my_y = lax.axis_index("y")
        my_z = lax.axis_index("z")
        r = my_x * Z + my_z
        my_lin = (my_x * Y + my_y) * Z + my_z

        start = r * rows
        cp_x = pltpu.make_async_copy(
            x_hbm.at[pl.ds(start, rows), :], xv, in_sems.at[0])
        cp_dy = pltpu.make_async_copy(
            dy_hbm.at[pl.ds(start, rows), :], dyv, in_sems.at[1])
        cp_x.start()
        cp_dy.start()

        barrier = pltpu.get_barrier_semaphore()
        for px in range(X):
            for py in range(Y):
                for pz in range(Z):
                    p_lin = (px * Y + py) * Z + pz

                    @pl.when(p_lin != my_lin)
                    def _():
                        pl.semaphore_signal(
                            barrier, inc=1,
                            device_id=(px, py, pz),
                            device_id_type=pl.DeviceIdType.MESH,
                        )
        pl.semaphore_wait(barrier, N_DEV - 1)

        cp_x.wait()
        cp_dy.wait()
        xb = xv[...]
        dyb = dyv[...]
        mu = jnp.mean(xb, axis=1, keepdims=True)
        var = jnp.mean(xb * xb, axis=1, keepdims=True) - mu * mu
        rstd = lax.rsqrt(var + 1e-5)
        xhat = (xb - mu) * rstd
        dgamma_p = jnp.sum(dyb * xhat, axis=0, keepdims=True)
        dbeta_p = jnp.sum(dyb, axis=0, keepdims=True)
        acc_ref[...] = jnp.concatenate([dgamma_p, dbeta_p], axis=0)

        loc = pltpu.make_async_copy(acc_ref, comm_ref.at[my_lin], loc_sem)
        loc.start()
        rdmas = []
        for px in range(X):
            for py in range(Y):
                for pz in range(Z):
                    p_lin = (px * Y + py) * Z + pz
                    rdma = pltpu.make_async_remote_copy(
                        src_ref=acc_ref,
                        dst_ref=comm_ref.at[my_lin],
                        send_sem=send_sems.at[p_lin],
                        recv_sem=recv_sems.at[my_lin],
                        device_id=(px, py, pz),
                        device_id_type=pl.DeviceIdType.MESH,
                    )
                    rdmas.append((p_lin, rdma))

                    @pl.when(p_lin != my_lin)
                    def _():
                        rdma.start()

        for px in range(X):
            for py in range(Y):
                for pz in range(Z):
                    p_lin = (px * Y + py) * Z + pz
                    recv = pltpu.make_async_remote_copy(
                        src_ref=acc_ref,
                        dst_ref=comm_ref.at[p_lin],
                        send_sem=send_sems.at[p_lin],
                        recv_sem=recv_sems.at[p_lin],
                        device_id=(px, py, pz),
                        device_id_type=pl.DeviceIdType.MESH,
                    )

                    @pl.when(p_lin != my_lin)
                    def _():
                        recv.wait_recv()

        loc.wait()
        out_ref[...] = jnp.sum(comm_ref[...], axis=0)

        for p_lin, rdma in rdmas:
            @pl.when(p_lin != my_lin)
            def _():
                rdma.wait_send()

    return pl.pallas_call(
        body,
        in_specs=[
            pl.BlockSpec(memory_space=pl.ANY),
            pl.BlockSpec(memory_space=pl.ANY),
        ],
        out_specs=pl.BlockSpec(memory_space=pltpu.VMEM),
        out_shape=jax.ShapeDtypeStruct((2, d), jnp.float32),
        scratch_shapes=[
            pltpu.VMEM((rows, d), jnp.float32),
            pltpu.VMEM((rows, d), jnp.float32),
            pltpu.VMEM((2, d), jnp.float32),
            pltpu.VMEM((N_DEV, 2, d), jnp.float32),
            pltpu.SemaphoreType.DMA((2,)),
            pltpu.SemaphoreType.DMA,
            pltpu.SemaphoreType.DMA((N_DEV,)),
            pltpu.SemaphoreType.DMA((N_DEV,)),
        ],
        compiler_params=pltpu.CompilerParams(
            collective_id=0,
        ),
    )(x, dy)
